# Initial kernel scaffold; baseline (speedup 1.0000x reference)
import jax
import jax.numpy as jnp
from jax.experimental import pallas as pl


def kernel(z, weight):
    raise NotImplementedError("write your pallas kernel here")


# speedup vs baseline: 124.7610x; 124.7610x over previous
import functools

import jax
import jax.numpy as jnp
from jax import lax
from jax.experimental import pallas as pl
from jax.experimental.pallas import tpu as pltpu
from jax.experimental.pallas import tpu_sc as plsc

N_CODES = 8192
DIM = 256
BETA = 0.25
ROW_TILE = 512

_NC, _NS = 2, 16
_NW = _NC * _NS


COL_CHUNK = 1024


def _dist_argmax_body(zn_ref, z2_ref, wt_ref, wn_ref, idx_ref):
    zneg = -zn_ref[...]
    z2 = z2_ref[...]
    run_m = jnp.full((ROW_TILE, 1), -jnp.inf, jnp.float32)
    run_i = jnp.zeros((ROW_TILE, 1), jnp.float32)
    col = lax.broadcasted_iota(
        jnp.int32, (ROW_TILE, COL_CHUNK), 1).astype(jnp.float32)
    for c in range(N_CODES // COL_CHUNK):
        sl = pl.ds(c * COL_CHUNK, COL_CHUNK)
        mm2 = jnp.dot(z2, wt_ref[:, sl], preferred_element_type=jnp.float32)
        d = (zneg - wn_ref[:, sl]) + mm2
        mc = jnp.max(d, axis=1, keepdims=True)
        bc = (jnp.max(jnp.where(d == mc, col, -1.0), axis=1, keepdims=True)
              + float(c * COL_CHUNK))
        upd = mc >= run_m
        run_i = jnp.where(upd, bc, run_i)
        run_m = jnp.where(upd, mc, run_m)
    idx_ref[...] = run_i[:, 0].astype(jnp.int32).reshape(1, 1, ROW_TILE)


def _dist_argmax(zn, z_flat, wt, wnorm):
    rows = z_flat.shape[0]
    nt = rows // ROW_TILE
    z2 = z_flat * 2.0
    out = pl.pallas_call(
        _dist_argmax_body,
        grid=(nt,),
        in_specs=[
            pl.BlockSpec((ROW_TILE, 1), lambda i: (i, 0)),
            pl.BlockSpec((ROW_TILE, DIM), lambda i: (i, 0)),
            pl.BlockSpec((DIM, N_CODES), lambda i: (0, 0)),
            pl.BlockSpec((1, N_CODES), lambda i: (0, 0)),
        ],
        out_specs=pl.BlockSpec((1, 1, ROW_TILE), lambda i: (i, 0, 0)),
        out_shape=jax.ShapeDtypeStruct((nt, 1, ROW_TILE), jnp.int32),
    )(zn, z2, wt, wnorm)
    return out.reshape(rows)


def _sc_gather_bincount(idx, weight, rows):
    rpw = rows // _NW
    half = rpw // 2
    mesh = plsc.VectorSubcoreMesh(core_axis_name="c", subcore_axis_name="s")

    @functools.partial(
        pl.kernel,
        mesh=mesh,
        out_type=[jax.ShapeDtypeStruct((rows, DIM), jnp.float32),
                  jax.ShapeDtypeStruct((_NC, N_CODES), jnp.float32)],
        scratch_types=[
            pltpu.VMEM((2, half), jnp.int32),
            pltpu.VMEM((rpw, DIM), jnp.float32),
            pltpu.VMEM((80,), jnp.float32),
            pltpu.VMEM((N_CODES // _NS,), jnp.float32),
            pltpu.VMEM_SHARED((N_CODES,), jnp.float32),
            pltpu.SemaphoreType.DMA,
        ],
    )
    def body(idx_hbm, w_hbm, zq_hbm, cnt_hbm, idx_v, rows_v, ones_v, zero_v,
             cnt_sh, sem):
        c = lax.axis_index("c")
        s = lax.axis_index("s")
        wid = c * _NS + s
        base = wid * rpw

        pltpu.sync_copy(idx_hbm.at[pl.ds(base, half)], idx_v.at[0])
        pltpu.sync_copy(idx_hbm.at[pl.ds(base + half, half)], idx_v.at[1])
        cp0 = pltpu.async_copy(w_hbm.at[idx_v.at[0]],
                               rows_v.at[pl.ds(0, half)], sem)
        cp1 = pltpu.async_copy(w_hbm.at[idx_v.at[1]],
                               rows_v.at[pl.ds(half, half)], sem)

        def _fill_ones(i, carry):
            ones_v[pl.ds(i * 16, 16)] = jnp.ones((16,), jnp.float32)
            return carry
        lax.fori_loop(0, 5, _fill_ones, 0)

        def _fill_zero(i, carry):
            zero_v[pl.ds(i * 16, 16)] = jnp.zeros((16,), jnp.float32)
            return carry
        lax.fori_loop(0, N_CODES // _NS // 16, _fill_zero, 0)
        pltpu.sync_copy(zero_v, cnt_sh.at[pl.ds(s * (N_CODES // _NS),
                                                N_CODES // _NS)])

        plsc.subcore_barrier()
        pltpu.sync_copy(ones_v.at[pl.ds(0, half)],
                        cnt_sh.at[idx_v.at[0]], add=True)
        pltpu.sync_copy(ones_v.at[pl.ds(0, half)],
                        cnt_sh.at[idx_v.at[1]], add=True)
        plsc.subcore_barrier()

        @pl.when(s == 0)
        def _():
            pltpu.sync_copy(cnt_sh, cnt_hbm.at[c])

        cp0.wait()
        cp1.wait()
        pltpu.sync_copy(rows_v, zq_hbm.at[pl.ds(base, rpw)])

    return body(idx, weight)


def _finalize_body(zq_ref, z_ref, cnt_ref, loss_ref, perp_ref):
    d = zq_ref[...] - z_ref[...]
    m = jnp.sum(d * d) * (1.0 / (zq_ref.shape[0] * zq_ref.shape[1]))
    loss_ref[...] = jnp.reshape(BETA * m + m, (1, 1))
    p = jnp.sum(cnt_ref[...], axis=0, keepdims=True) * (1.0 / zq_ref.shape[0])
    ent = jnp.sum(p * jnp.log(p + 1e-10))
    perp_ref[...] = jnp.reshape(jnp.exp(-ent), (1, 1))


def _finalize(zq_flat, z_flat, cnt):
    return pl.pallas_call(
        _finalize_body,
        out_shape=[jax.ShapeDtypeStruct((1, 1), jnp.float32),
                   jax.ShapeDtypeStruct((1, 1), jnp.float32)],
    )(zq_flat, z_flat, cnt)


def kernel(z, weight):
    b, cdim, h, w = z.shape
    zp = jnp.transpose(z, (0, 2, 3, 1))
    z_flat = zp.reshape(-1, cdim)
    rows = z_flat.shape[0]
    wt = weight.T
    zn = jnp.sum(z_flat ** 2, axis=1, keepdims=True)
    wnorm = jnp.sum(weight ** 2, axis=1)[None, :]
    idx = _dist_argmax(zn, z_flat, wt, wnorm)
    zq_flat, cnt = _sc_gather_bincount(idx, weight, rows)
    loss, perp = _finalize(zq_flat, z_flat, cnt)
    zq_out = jnp.transpose(zq_flat.reshape(zp.shape), (0, 3, 1, 2))
    return zq_out, loss.reshape(()), perp.reshape(()), idx


# speedup vs baseline: 126.3183x; 1.0125x over previous
import functools

import jax
import jax.numpy as jnp
from jax import lax
from jax.experimental import pallas as pl
from jax.experimental.pallas import tpu as pltpu
from jax.experimental.pallas import tpu_sc as plsc

N_CODES = 8192
DIM = 256
BETA = 0.25
ROW_TILE = 512

_NC, _NS = 2, 16
_NW = _NC * _NS


COL_CHUNK = 1024


def _dist_argmax_body(zn_ref, z2_ref, wt_ref, wn_ref, idx_ref):
    zneg = -zn_ref[...]
    z2 = z2_ref[...] * 2.0
    run_m = jnp.full((ROW_TILE, 1), -jnp.inf, jnp.float32)
    run_i = jnp.zeros((ROW_TILE, 1), jnp.float32)
    col = lax.broadcasted_iota(
        jnp.int32, (ROW_TILE, COL_CHUNK), 1).astype(jnp.float32)
    for c in range(N_CODES // COL_CHUNK):
        sl = pl.ds(c * COL_CHUNK, COL_CHUNK)
        mm2 = jnp.dot(z2, wt_ref[:, sl], preferred_element_type=jnp.float32)
        d = (zneg - wn_ref[:, sl]) + mm2
        mc = jnp.max(d, axis=1, keepdims=True)
        bc = (jnp.max(jnp.where(d == mc, col, -1.0), axis=1, keepdims=True)
              + float(c * COL_CHUNK))
        upd = mc >= run_m
        run_i = jnp.where(upd, bc, run_i)
        run_m = jnp.where(upd, mc, run_m)
    idx_ref[...] = run_i[:, 0].astype(jnp.int32).reshape(1, 1, ROW_TILE)


def _dist_argmax(zn, z_flat, wt, wnorm):
    rows = z_flat.shape[0]
    nt = rows // ROW_TILE
    out = pl.pallas_call(
        _dist_argmax_body,
        grid=(nt,),
        in_specs=[
            pl.BlockSpec((ROW_TILE, 1), lambda i: (i, 0)),
            pl.BlockSpec((ROW_TILE, DIM), lambda i: (i, 0)),
            pl.BlockSpec((DIM, N_CODES), lambda i: (0, 0)),
            pl.BlockSpec((1, N_CODES), lambda i: (0, 0)),
        ],
        out_specs=pl.BlockSpec((1, 1, ROW_TILE), lambda i: (i, 0, 0)),
        out_shape=jax.ShapeDtypeStruct((nt, 1, ROW_TILE), jnp.int32),
    )(zn, z_flat, wt, wnorm)
    return out.reshape(rows)


def _sc_gather_bincount(idx, weight, rows):
    rpw = rows // _NW
    half = rpw // 2
    mesh = plsc.VectorSubcoreMesh(core_axis_name="c", subcore_axis_name="s")

    @functools.partial(
        pl.kernel,
        mesh=mesh,
        out_type=[jax.ShapeDtypeStruct((rows, DIM), jnp.float32),
                  jax.ShapeDtypeStruct((_NC, N_CODES), jnp.float32)],
        scratch_types=[
            pltpu.VMEM((2, half), jnp.int32),
            pltpu.VMEM((rpw, DIM), jnp.float32),
            pltpu.VMEM((80,), jnp.float32),
            pltpu.VMEM((N_CODES // _NS,), jnp.float32),
            pltpu.VMEM_SHARED((N_CODES,), jnp.float32),
            pltpu.SemaphoreType.DMA,
        ],
    )
    def body(idx_hbm, w_hbm, zq_hbm, cnt_hbm, idx_v, rows_v, ones_v, zero_v,
             cnt_sh, sem):
        c = lax.axis_index("c")
        s = lax.axis_index("s")
        wid = c * _NS + s
        base = wid * rpw

        pltpu.sync_copy(idx_hbm.at[pl.ds(base, half)], idx_v.at[0])
        pltpu.sync_copy(idx_hbm.at[pl.ds(base + half, half)], idx_v.at[1])
        cp0 = pltpu.async_copy(w_hbm.at[idx_v.at[0]],
                               rows_v.at[pl.ds(0, half)], sem)
        cp1 = pltpu.async_copy(w_hbm.at[idx_v.at[1]],
                               rows_v.at[pl.ds(half, half)], sem)

        def _fill_ones(i, carry):
            ones_v[pl.ds(i * 16, 16)] = jnp.ones((16,), jnp.float32)
            return carry
        lax.fori_loop(0, 5, _fill_ones, 0)

        def _fill_zero(i, carry):
            zero_v[pl.ds(i * 16, 16)] = jnp.zeros((16,), jnp.float32)
            return carry
        lax.fori_loop(0, N_CODES // _NS // 16, _fill_zero, 0)
        pltpu.sync_copy(zero_v, cnt_sh.at[pl.ds(s * (N_CODES // _NS),
                                                N_CODES // _NS)])

        plsc.subcore_barrier()
        pltpu.sync_copy(ones_v.at[pl.ds(0, half)],
                        cnt_sh.at[idx_v.at[0]], add=True)
        pltpu.sync_copy(ones_v.at[pl.ds(0, half)],
                        cnt_sh.at[idx_v.at[1]], add=True)
        plsc.subcore_barrier()

        @pl.when(s == 0)
        def _():
            pltpu.sync_copy(cnt_sh, cnt_hbm.at[c])

        cp0.wait()
        cp1.wait()
        pltpu.sync_copy(rows_v, zq_hbm.at[pl.ds(base, rpw)])

    return body(idx, weight)


def _finalize_body(zq_ref, z_ref, cnt_ref, loss_ref, perp_ref):
    d = zq_ref[...] - z_ref[...]
    m = jnp.sum(d * d) * (1.0 / (zq_ref.shape[0] * zq_ref.shape[1]))
    loss_ref[...] = jnp.reshape(BETA * m + m, (1, 1))
    p = jnp.sum(cnt_ref[...], axis=0, keepdims=True) * (1.0 / zq_ref.shape[0])
    ent = jnp.sum(p * jnp.log(p + 1e-10))
    perp_ref[...] = jnp.reshape(jnp.exp(-ent), (1, 1))


def _finalize(zq_flat, z_flat, cnt):
    return pl.pallas_call(
        _finalize_body,
        out_shape=[jax.ShapeDtypeStruct((1, 1), jnp.float32),
                   jax.ShapeDtypeStruct((1, 1), jnp.float32)],
    )(zq_flat, z_flat, cnt)


def kernel(z, weight):
    b, cdim, h, w = z.shape
    zp = jnp.transpose(z, (0, 2, 3, 1))
    z_flat = zp.reshape(-1, cdim)
    rows = z_flat.shape[0]
    wt = weight.T
    zn = jnp.sum(z_flat ** 2, axis=1, keepdims=True)
    wnorm = jnp.sum(weight ** 2, axis=1)[None, :]
    idx = _dist_argmax(zn, z_flat, wt, wnorm)
    zq_flat, cnt = _sc_gather_bincount(idx, weight, rows)
    loss, perp = _finalize(zq_flat, z_flat, cnt)
    zq_out = jnp.transpose(zq_flat.reshape(zp.shape), (0, 3, 1, 2))
    return zq_out, loss.reshape(()), perp.reshape(()), idx


# speedup vs baseline: 136.0421x; 1.0770x over previous
import functools

import jax
import jax.numpy as jnp
from jax import lax
from jax.experimental import pallas as pl
from jax.experimental.pallas import tpu as pltpu
from jax.experimental.pallas import tpu_sc as plsc

N_CODES = 8192
DIM = 256
BETA = 0.25
ROW_TILE = 512

_NC, _NS = 2, 16
_NW = _NC * _NS


COL_CHUNK = 1024


def _dist_argmax_body(zn_ref, z2_ref, wt_ref, wn_ref, idx_ref):
    zneg = -zn_ref[...]
    z2 = z2_ref[...] * 2.0
    run_m = jnp.full((ROW_TILE, 1), -jnp.inf, jnp.float32)
    run_i = jnp.zeros((ROW_TILE, 1), jnp.float32)
    col = lax.broadcasted_iota(
        jnp.int32, (ROW_TILE, COL_CHUNK), 1).astype(jnp.float32)
    for c in range(N_CODES // COL_CHUNK):
        sl = pl.ds(c * COL_CHUNK, COL_CHUNK)
        mm2 = lax.dot_general(z2, wt_ref[sl, :], (((1,), (1,)), ((), ())),
                              preferred_element_type=jnp.float32)
        d = (zneg - wn_ref[:, pl.ds(c * COL_CHUNK, COL_CHUNK)]) + mm2
        mc = jnp.max(d, axis=1, keepdims=True)
        bc = (jnp.max(jnp.where(d == mc, col, -1.0), axis=1, keepdims=True)
              + float(c * COL_CHUNK))
        upd = mc >= run_m
        run_i = jnp.where(upd, bc, run_i)
        run_m = jnp.where(upd, mc, run_m)
    idx_ref[...] = run_i[:, 0].astype(jnp.int32).reshape(1, 1, ROW_TILE)


def _dist_argmax(zn, z_flat, w, wnorm):
    rows = z_flat.shape[0]
    nt = rows // ROW_TILE
    out = pl.pallas_call(
        _dist_argmax_body,
        grid=(nt,),
        in_specs=[
            pl.BlockSpec((ROW_TILE, 1), lambda i: (i, 0)),
            pl.BlockSpec((ROW_TILE, DIM), lambda i: (i, 0)),
            pl.BlockSpec((N_CODES, DIM), lambda i: (0, 0)),
            pl.BlockSpec((1, N_CODES), lambda i: (0, 0)),
        ],
        out_specs=pl.BlockSpec((1, 1, ROW_TILE), lambda i: (i, 0, 0)),
        out_shape=jax.ShapeDtypeStruct((nt, 1, ROW_TILE), jnp.int32),
    )(zn, z_flat, w, wnorm)
    return out.reshape(rows)


def _sc_gather_bincount(idx, weight, rows):
    rpw = rows // _NW
    half = rpw // 2
    mesh = plsc.VectorSubcoreMesh(core_axis_name="c", subcore_axis_name="s")

    @functools.partial(
        pl.kernel,
        mesh=mesh,
        out_type=[jax.ShapeDtypeStruct((rows, DIM), jnp.float32),
                  jax.ShapeDtypeStruct((_NC, N_CODES), jnp.float32)],
        scratch_types=[
            pltpu.VMEM((2, half), jnp.int32),
            pltpu.VMEM((rpw, DIM), jnp.float32),
            pltpu.VMEM((80,), jnp.float32),
            pltpu.VMEM((N_CODES // _NS,), jnp.float32),
            pltpu.VMEM_SHARED((N_CODES,), jnp.float32),
            pltpu.SemaphoreType.DMA,
        ],
    )
    def body(idx_hbm, w_hbm, zq_hbm, cnt_hbm, idx_v, rows_v, ones_v, zero_v,
             cnt_sh, sem):
        c = lax.axis_index("c")
        s = lax.axis_index("s")
        wid = c * _NS + s
        base = wid * rpw

        pltpu.sync_copy(idx_hbm.at[pl.ds(base, half)], idx_v.at[0])
        pltpu.sync_copy(idx_hbm.at[pl.ds(base + half, half)], idx_v.at[1])
        cp0 = pltpu.async_copy(w_hbm.at[idx_v.at[0]],
                               rows_v.at[pl.ds(0, half)], sem)
        cp1 = pltpu.async_copy(w_hbm.at[idx_v.at[1]],
                               rows_v.at[pl.ds(half, half)], sem)

        def _fill_ones(i, carry):
            ones_v[pl.ds(i * 16, 16)] = jnp.ones((16,), jnp.float32)
            return carry
        lax.fori_loop(0, 5, _fill_ones, 0)

        def _fill_zero(i, carry):
            zero_v[pl.ds(i * 16, 16)] = jnp.zeros((16,), jnp.float32)
            return carry
        lax.fori_loop(0, N_CODES // _NS // 16, _fill_zero, 0)
        pltpu.sync_copy(zero_v, cnt_sh.at[pl.ds(s * (N_CODES // _NS),
                                                N_CODES // _NS)])

        plsc.subcore_barrier()
        pltpu.sync_copy(ones_v.at[pl.ds(0, half)],
                        cnt_sh.at[idx_v.at[0]], add=True)
        pltpu.sync_copy(ones_v.at[pl.ds(0, half)],
                        cnt_sh.at[idx_v.at[1]], add=True)
        plsc.subcore_barrier()

        @pl.when(s == 0)
        def _():
            pltpu.sync_copy(cnt_sh, cnt_hbm.at[c])

        cp0.wait()
        cp1.wait()
        pltpu.sync_copy(rows_v, zq_hbm.at[pl.ds(base, rpw)])

    return body(idx, weight)


def _finalize_body(zq_ref, z_ref, cnt_ref, loss_ref, perp_ref):
    d = zq_ref[...] - z_ref[...]
    m = jnp.sum(d * d) * (1.0 / (zq_ref.shape[0] * zq_ref.shape[1]))
    loss_ref[...] = jnp.reshape(BETA * m + m, (1, 1))
    p = jnp.sum(cnt_ref[...], axis=0, keepdims=True) * (1.0 / zq_ref.shape[0])
    ent = jnp.sum(p * jnp.log(p + 1e-10))
    perp_ref[...] = jnp.reshape(jnp.exp(-ent), (1, 1))


def _finalize(zq_flat, z_flat, cnt):
    return pl.pallas_call(
        _finalize_body,
        out_shape=[jax.ShapeDtypeStruct((1, 1), jnp.float32),
                   jax.ShapeDtypeStruct((1, 1), jnp.float32)],
    )(zq_flat, z_flat, cnt)


def kernel(z, weight):
    b, cdim, h, w = z.shape
    zp = jnp.transpose(z, (0, 2, 3, 1))
    z_flat = zp.reshape(-1, cdim)
    rows = z_flat.shape[0]
    zn = jnp.sum(z_flat ** 2, axis=1, keepdims=True)
    wnorm = jnp.sum(weight ** 2, axis=1)[None, :]
    idx = _dist_argmax(zn, z_flat, weight, wnorm)
    zq_flat, cnt = _sc_gather_bincount(idx, weight, rows)
    loss, perp = _finalize(zq_flat, z_flat, cnt)
    zq_out = jnp.transpose(zq_flat.reshape(zp.shape), (0, 3, 1, 2))
    return zq_out, loss.reshape(()), perp.reshape(()), idx
